# baseline (speedup 1.0000x reference)
import functools

import jax
import jax.numpy as jnp
from jax.experimental import pallas as pl

_INPUT_SIZE = 4
_N_CLASSES = 16
_HIDDEN = 100
_PAD_VALUE = -100.0


def _mlp_kernel(x_ref, w1t_ref, b1_ref, w2t_ref, b2_ref, o_ref, *, t_tile):
    j = pl.program_id(1)
    xs = x_ref[0, :, pl.ds(j * t_tile, t_tile + _INPUT_SIZE - 1)]
    y = jnp.concatenate(
        [xs[:, k:k + t_tile] for k in range(_INPUT_SIZE)], axis=0)
    h = jnp.dot(w1t_ref[:, :], y, preferred_element_type=jnp.float32)
    h = jnp.maximum(h + b1_ref[:, :], 0.0)
    logits = jnp.dot(w2t_ref[:, :], h, preferred_element_type=jnp.float32)
    logits = logits + b2_ref[:, :]
    m = jnp.max(logits, axis=0, keepdims=True)
    shifted = logits - m
    lse = jnp.log(jnp.sum(jnp.exp(shifted), axis=0, keepdims=True))
    o_ref[0, :, :] = jnp.transpose(shifted - lse)


@jax.jit
def kernel(input_, W1, b1, W2, b2):
    B, T = input_.shape
    TT = 8192
    left = jnp.full((B, _INPUT_SIZE - 1), _PAD_VALUE, input_.dtype)
    right = jnp.zeros((B, 128 - (_INPUT_SIZE - 1)), input_.dtype)
    xp = jnp.concatenate([left, input_, right], axis=1)
    xp = xp.reshape(B, 1, xp.shape[1])
    out = pl.pallas_call(
        functools.partial(_mlp_kernel, t_tile=TT),
        grid=(B, T // TT),
        in_specs=[
            pl.BlockSpec((1, 1, xp.shape[2]), lambda i, j: (i, 0, 0)),
            pl.BlockSpec((_HIDDEN, _INPUT_SIZE), lambda i, j: (0, 0)),
            pl.BlockSpec((_HIDDEN, 1), lambda i, j: (0, 0)),
            pl.BlockSpec((_N_CLASSES, _HIDDEN), lambda i, j: (0, 0)),
            pl.BlockSpec((_N_CLASSES, 1), lambda i, j: (0, 0)),
        ],
        out_specs=pl.BlockSpec((1, TT, _N_CLASSES), lambda i, j: (i, j, 0)),
        out_shape=jax.ShapeDtypeStruct((B, T, _N_CLASSES), jnp.float32),
    )(xp, W1.T, b1.reshape(_HIDDEN, 1), W2.T, b2.reshape(_N_CLASSES, 1))
    return out


# speedup vs baseline: 1.0404x; 1.0404x over previous
import functools

import jax
import jax.numpy as jnp
from jax.experimental import pallas as pl
from jax.experimental.pallas import tpu as pltpu

_INPUT_SIZE = 4
_N_CLASSES = 16
_HIDDEN = 100
_PAD_VALUE = -100.0


def _mlp_kernel(x_ref, w1t_ref, b1_ref, w2t_ref, b2_ref, o_ref, *, t_tile):
    j = pl.program_id(1)
    xs = x_ref[0, :, pl.ds(j * t_tile, t_tile + _INPUT_SIZE - 1)]
    y = jnp.concatenate(
        [xs[:, k:k + t_tile] for k in range(_INPUT_SIZE)], axis=0)
    h = jnp.dot(w1t_ref[:, :], y, preferred_element_type=jnp.float32)
    h = jnp.maximum(h + b1_ref[:, :], 0.0)
    logits = jnp.dot(w2t_ref[:, :], h, preferred_element_type=jnp.float32)
    logits = logits + b2_ref[:, :]
    m = jnp.max(logits, axis=0, keepdims=True)
    shifted = logits - m
    lse = jnp.log(jnp.sum(jnp.exp(shifted), axis=0, keepdims=True))
    res = jnp.transpose(shifted - lse)
    resw = jnp.concatenate(
        [res, jnp.zeros((t_tile, 128 - _N_CLASSES), jnp.float32)], axis=1)
    rolled = pltpu.roll(resw, 0, 1, stride=_N_CLASSES, stride_axis=0)
    o_ref[0, :, :] = rolled.reshape(t_tile // 8, 8, 128).sum(axis=1)


@jax.jit
def kernel(input_, W1, b1, W2, b2):
    B, T = input_.shape
    TT = 8192
    left = jnp.full((B, _INPUT_SIZE - 1), _PAD_VALUE, input_.dtype)
    right = jnp.zeros((B, 128 - (_INPUT_SIZE - 1)), input_.dtype)
    xp = jnp.concatenate([left, input_, right], axis=1)
    xp = xp.reshape(B, 1, xp.shape[1])
    out = pl.pallas_call(
        functools.partial(_mlp_kernel, t_tile=TT),
        grid=(B, T // TT),
        in_specs=[
            pl.BlockSpec((1, 1, xp.shape[2]), lambda i, j: (i, 0, 0)),
            pl.BlockSpec((_HIDDEN, _INPUT_SIZE), lambda i, j: (0, 0)),
            pl.BlockSpec((_HIDDEN, 1), lambda i, j: (0, 0)),
            pl.BlockSpec((_N_CLASSES, _HIDDEN), lambda i, j: (0, 0)),
            pl.BlockSpec((_N_CLASSES, 1), lambda i, j: (0, 0)),
        ],
        out_specs=pl.BlockSpec((1, TT // 8, 128), lambda i, j: (i, j, 0)),
        out_shape=jax.ShapeDtypeStruct((B, T * _N_CLASSES // 128, 128),
                                       jnp.float32),
    )(xp, W1.T, b1.reshape(_HIDDEN, 1), W2.T, b2.reshape(_N_CLASSES, 1))
    return out.reshape(B, T, _N_CLASSES)
